# baseline (speedup 1.0000x reference)
import functools

import jax
import jax.numpy as jnp
from jax import lax
from jax.experimental import pallas as pl
from jax.experimental.pallas import tpu as pltpu
from jax.experimental.pallas import tpu_sc as plsc

B_ = 16384
F_ = 26
VOCAB_ = 100000
EMB_ = 32
DM_ = 128
ROWS = B_ * F_
NW = 32
BW = B_ // NW


TROW = F_ * VOCAB_
TCH = 1024
NCH = TROW // TCH
TAIL = TROW - NCH * TCH
CPW = (NCH + NW - 1) // NW


@functools.cache
def _make_sc_transpose():
    mesh = plsc.VectorSubcoreMesh(core_axis_name="c", subcore_axis_name="s")

    @functools.partial(
        pl.kernel,
        out_type=jax.ShapeDtypeStruct((TROW * EMB_ // DM_, DM_), jnp.float32),
        mesh=mesh,
        compiler_params=pltpu.CompilerParams(needs_layout_passes=False),
        scratch_types=[
            pltpu.VMEM((EMB_, TCH), jnp.float32),
            pltpu.VMEM((TCH // 4, DM_), jnp.float32),
            pltpu.VMEM((TAIL, EMB_), jnp.float32),
        ],
    )
    def sc_transpose(tableT, tail64, out, in_v, out_v, tail_v):
        wid = lax.axis_index("s") * 2 + lax.axis_index("c")

        def transpose_slab(nrows):
            def row_loop(r, carry):
                for h in range(DM_ // 16):
                    u = h // 2
                    e0 = (h % 2) * 16
                    vals = plsc.load_gather(
                        in_v,
                        [lax.iota(jnp.int32, 16) + e0,
                         jnp.full((16,), 0, jnp.int32) + (4 * r + u)],
                    )
                    out_v[r, pl.ds(h * 16, 16)] = vals
                return carry
            lax.fori_loop(0, nrows, row_loop, 0)

        def do_chunk(k, carry):
            cid = wid + NW * k

            @pl.when(cid < NCH)
            def _():
                pltpu.sync_copy(tableT.at[:, pl.ds(cid * TCH, TCH)], in_v)
                transpose_slab(TCH // 4)
                pltpu.sync_copy(
                    out_v, out.at[pl.ds(cid * (TCH // 4), TCH // 4)]
                )
            return carry

        lax.fori_loop(0, CPW, do_chunk, 0)

        @pl.when(wid == 0)
        def _tail():
            pltpu.sync_copy(tail64, tail_v)

            def tail_row(r, carry):
                for h in range(DM_ // 16):
                    u = h // 2
                    e0 = (h % 2) * 16
                    vals = plsc.load_gather(
                        tail_v,
                        [jnp.full((16,), 0, jnp.int32) + (4 * r + u),
                         lax.iota(jnp.int32, 16) + e0],
                    )
                    out_v[r, pl.ds(h * 16, 16)] = vals
                return carry

            lax.fori_loop(0, TAIL // 4, tail_row, 0)
            pltpu.sync_copy(
                out_v.at[pl.ds(0, TAIL // 4)],
                out.at[pl.ds(NCH * (TCH // 4), TAIL // 4)],
            )

    return sc_transpose


@functools.cache
def _make_sc_gather():
    mesh = plsc.VectorSubcoreMesh(core_axis_name="c", subcore_axis_name="s")

    @functools.partial(
        pl.kernel,
        out_type=jax.ShapeDtypeStruct((ROWS, EMB_), jnp.float32),
        mesh=mesh,
        compiler_params=pltpu.CompilerParams(use_tc_tiling_on_sc=False),
        scratch_types=[
            pltpu.VMEM((F_, BW), jnp.int32),
            pltpu.VMEM((F_ * BW,), jnp.int32),
            pltpu.VMEM((BW, EMB_), jnp.float32),
            pltpu.VMEM((BW, EMB_), jnp.float32),
            pltpu.SemaphoreType.DMA,
            pltpu.SemaphoreType.DMA,
        ],
    )
    def sc_gather(table, featsT, out, feats_v, idx_v, buf0, buf1, sem0, sem1):
        wid = lax.axis_index("s") * 2 + lax.axis_index("c")
        pltpu.sync_copy(featsT.at[:, pl.ds(wid * BW, BW)], feats_v)

        def compute_idx(f, carry):
            off = f * VOCAB_
            for c in range(BW // 16):
                idx_v[pl.ds(f * BW + c * 16, 16)] = (
                    feats_v[f, pl.ds(c * 16, 16)] + off
                )
            return carry

        lax.fori_loop(0, F_, compute_idx, 0)

        bufs = (buf0, buf1)
        sems = (sem0, sem1)
        handles = [None] * F_

        def start(f):
            return pltpu.async_copy(
                table.at[idx_v.at[pl.ds(f * BW, BW)]], bufs[f % 2], sems[f % 2]
            )

        handles[0] = start(0)
        for f in range(F_):
            if f + 1 < F_:
                handles[f + 1] = start(f + 1)
            handles[f].wait()
            pltpu.sync_copy(bufs[f % 2], out.at[pl.ds(f * B_ + wid * BW, BW)])

    return sc_gather


BSB = 2048
NBB = B_ // BSB


def _tc_body(g_ref, w_ref, b_ref, gam_ref, bet_ref, o_ref):
    x = g_ref[...]
    h = jnp.dot(x, w_ref[...], preferred_element_type=jnp.float32) + b_ref[...]
    h = h / (1.0 + jnp.exp(-h))
    mu = jnp.mean(h, axis=1, keepdims=True)
    d = h - mu
    var = jnp.mean(d * d, axis=1, keepdims=True)
    y = d * lax.rsqrt(var + 1e-5)
    o_ref[...] = (y * gam_ref[...] + bet_ref[...]).reshape(1, BSB, DM_)


_tc_call = pl.pallas_call(
    _tc_body,
    grid=(F_, NBB),
    in_specs=[
        pl.BlockSpec((BSB, EMB_), lambda f, i: (f * NBB + i, 0)),
        pl.BlockSpec((EMB_, DM_), lambda f, i: (0, 0)),
        pl.BlockSpec((1, DM_), lambda f, i: (0, 0)),
        pl.BlockSpec((1, DM_), lambda f, i: (0, 0)),
        pl.BlockSpec((1, DM_), lambda f, i: (0, 0)),
    ],
    out_specs=pl.BlockSpec((1, BSB, DM_), lambda f, i: (f, i, 0)),
    out_shape=jax.ShapeDtypeStruct((F_, B_, DM_), jnp.float32),
)


def kernel(int_feats, missing_mask, emb_table, missing_embeddings, W, b, gamma, beta):
    featsT = int_feats.T
    tableT = emb_table.T
    tail64 = emb_table[NCH * TCH:]
    table_rm = _make_sc_transpose()(tableT, tail64)
    table_lin = table_rm.reshape(TROW, EMB_)
    g = _make_sc_gather()(table_lin, featsT)
    out3 = _tc_call(
        g, W, b.reshape(1, DM_), gamma.reshape(1, DM_), beta.reshape(1, DM_),
    )
    return jnp.transpose(out3, (1, 0, 2))


# speedup vs baseline: 1.1589x; 1.1589x over previous
import functools

import jax
import jax.numpy as jnp
from jax import lax
from jax.experimental import pallas as pl
from jax.experimental.pallas import tpu as pltpu
from jax.experimental.pallas import tpu_sc as plsc

B_ = 16384
F_ = 26
VOCAB_ = 100000
EMB_ = 32
DM_ = 128
ROWS = B_ * F_
NW = 32
BW = B_ // NW


TROW = F_ * VOCAB_
TCH = 1024
NCH = TROW // TCH
TAIL = TROW - NCH * TCH
CPW = (NCH + NW - 1) // NW


@functools.cache
def _make_sc_transpose():
    mesh = plsc.VectorSubcoreMesh(core_axis_name="c", subcore_axis_name="s")

    @functools.partial(
        pl.kernel,
        out_type=jax.ShapeDtypeStruct((TROW * EMB_ // DM_, DM_), jnp.float32),
        mesh=mesh,
        compiler_params=pltpu.CompilerParams(needs_layout_passes=False),
        scratch_types=[
            pltpu.VMEM((EMB_, TCH), jnp.float32),
            pltpu.VMEM((TCH // 4, DM_), jnp.float32),
            pltpu.VMEM((TAIL, EMB_), jnp.float32),
        ],
    )
    def sc_transpose(tableT, tail64, out, in_v, out_v, tail_v):
        wid = lax.axis_index("s") * 2 + lax.axis_index("c")

        def do_chunk(k, carry):
            cid = wid + NW * k

            @pl.when(cid < NCH)
            def _():
                pltpu.sync_copy(tableT.at[:, pl.ds(cid * TCH, TCH)], in_v)

                def col_group(cg, carry2):
                    c = lax.iota(jnp.int32, 16) + cg * 16
                    row_v = lax.shift_right_logical(c, 2)
                    col_v = (c & 3) * 32
                    for e in range(EMB_):
                        plsc.store_scatter(
                            out_v,
                            [row_v, col_v + e],
                            in_v[e, pl.ds(cg * 16, 16)],
                        )
                    return carry2

                lax.fori_loop(0, TCH // 16, col_group, 0)
                pltpu.sync_copy(
                    out_v, out.at[pl.ds(cid * (TCH // 4), TCH // 4)]
                )
            return carry

        lax.fori_loop(0, CPW, do_chunk, 0)

        @pl.when(wid == 0)
        def _tail():
            pltpu.sync_copy(tail64, tail_v)

            def tail_row(t, carry):
                r = lax.shift_right_logical(t, 2)
                u = t & 3
                for k in range(EMB_ // 16):
                    out_v[r, pl.ds(u * 32 + k * 16, 16)] = (
                        tail_v[t, pl.ds(k * 16, 16)]
                    )
                return carry

            lax.fori_loop(0, TAIL, tail_row, 0)
            pltpu.sync_copy(
                out_v.at[pl.ds(0, TAIL // 4)],
                out.at[pl.ds(NCH * (TCH // 4), TAIL // 4)],
            )

    return sc_transpose


@functools.cache
def _make_sc_gather():
    mesh = plsc.VectorSubcoreMesh(core_axis_name="c", subcore_axis_name="s")

    @functools.partial(
        pl.kernel,
        out_type=jax.ShapeDtypeStruct((ROWS, EMB_), jnp.float32),
        mesh=mesh,
        compiler_params=pltpu.CompilerParams(use_tc_tiling_on_sc=False),
        scratch_types=[
            pltpu.VMEM((F_, BW), jnp.int32),
            pltpu.VMEM((F_ * BW,), jnp.int32),
            pltpu.VMEM((BW, EMB_), jnp.float32),
            pltpu.VMEM((BW, EMB_), jnp.float32),
            pltpu.SemaphoreType.DMA,
            pltpu.SemaphoreType.DMA,
        ],
    )
    def sc_gather(table, featsT, out, feats_v, idx_v, buf0, buf1, sem0, sem1):
        wid = lax.axis_index("s") * 2 + lax.axis_index("c")
        pltpu.sync_copy(featsT.at[:, pl.ds(wid * BW, BW)], feats_v)

        def compute_idx(f, carry):
            off = f * VOCAB_
            for c in range(BW // 16):
                idx_v[pl.ds(f * BW + c * 16, 16)] = (
                    feats_v[f, pl.ds(c * 16, 16)] + off
                )
            return carry

        lax.fori_loop(0, F_, compute_idx, 0)

        bufs = (buf0, buf1)
        sems = (sem0, sem1)
        handles = [None] * F_

        def start(f):
            return pltpu.async_copy(
                table.at[idx_v.at[pl.ds(f * BW, BW)]], bufs[f % 2], sems[f % 2]
            )

        handles[0] = start(0)
        for f in range(F_):
            if f + 1 < F_:
                handles[f + 1] = start(f + 1)
            handles[f].wait()
            pltpu.sync_copy(bufs[f % 2], out.at[pl.ds(f * B_ + wid * BW, BW)])

    return sc_gather


BSB = 2048
NBB = B_ // BSB


def _tc_body(g_ref, w_ref, b_ref, gam_ref, bet_ref, o_ref):
    x = g_ref[...]
    h = jnp.dot(x, w_ref[...], preferred_element_type=jnp.float32) + b_ref[...]
    h = h / (1.0 + jnp.exp(-h))
    mu = jnp.mean(h, axis=1, keepdims=True)
    d = h - mu
    var = jnp.mean(d * d, axis=1, keepdims=True)
    y = d * lax.rsqrt(var + 1e-5)
    o_ref[...] = (y * gam_ref[...] + bet_ref[...]).reshape(1, BSB, DM_)


_tc_call = pl.pallas_call(
    _tc_body,
    grid=(F_, NBB),
    in_specs=[
        pl.BlockSpec((BSB, EMB_), lambda f, i: (f * NBB + i, 0)),
        pl.BlockSpec((EMB_, DM_), lambda f, i: (0, 0)),
        pl.BlockSpec((1, DM_), lambda f, i: (0, 0)),
        pl.BlockSpec((1, DM_), lambda f, i: (0, 0)),
        pl.BlockSpec((1, DM_), lambda f, i: (0, 0)),
    ],
    out_specs=pl.BlockSpec((1, BSB, DM_), lambda f, i: (f, i, 0)),
    out_shape=jax.ShapeDtypeStruct((F_, B_, DM_), jnp.float32),
)


def kernel(int_feats, missing_mask, emb_table, missing_embeddings, W, b, gamma, beta):
    featsT = int_feats.T
    tableT = emb_table.T
    tail64 = emb_table[NCH * TCH:]
    table_rm = _make_sc_transpose()(tableT, tail64)
    table_lin = table_rm.reshape(TROW, EMB_)
    g = _make_sc_gather()(table_lin, featsT)
    out3 = _tc_call(
        g, W, b.reshape(1, DM_), gamma.reshape(1, DM_), beta.reshape(1, DM_),
    )
    return jnp.transpose(out3, (1, 0, 2))


# speedup vs baseline: 1.3738x; 1.1854x over previous
import functools

import jax
import jax.numpy as jnp
from jax import lax
from jax.experimental import pallas as pl
from jax.experimental.pallas import tpu as pltpu
from jax.experimental.pallas import tpu_sc as plsc

B_ = 16384
F_ = 26
VOCAB_ = 100000
EMB_ = 32
DM_ = 128
ROWS = B_ * F_
NW = 32
BW = B_ // NW


@functools.cache
def _make_sc_gather():
    mesh = plsc.VectorSubcoreMesh(core_axis_name="c", subcore_axis_name="s")

    @functools.partial(
        pl.kernel,
        out_type=jax.ShapeDtypeStruct((ROWS, EMB_), jnp.bfloat16),
        mesh=mesh,
        compiler_params=pltpu.CompilerParams(use_tc_tiling_on_sc=False),
        scratch_types=[
            pltpu.VMEM((F_, BW), jnp.int32),
            pltpu.VMEM((F_ * BW,), jnp.int32),
            pltpu.VMEM((BW, EMB_), jnp.bfloat16),
            pltpu.VMEM((BW, EMB_), jnp.bfloat16),
            pltpu.SemaphoreType.DMA,
            pltpu.SemaphoreType.DMA,
        ],
    )
    def sc_gather(table, featsT, out, feats_v, idx_v, buf0, buf1, sem0, sem1):
        wid = lax.axis_index("s") * 2 + lax.axis_index("c")
        pltpu.sync_copy(featsT.at[:, pl.ds(wid * BW, BW)], feats_v)

        def compute_idx(f, carry):
            off = f * VOCAB_
            for c in range(BW // 16):
                idx_v[pl.ds(f * BW + c * 16, 16)] = (
                    feats_v[f, pl.ds(c * 16, 16)] + off
                )
            return carry

        lax.fori_loop(0, F_, compute_idx, 0)

        bufs = (buf0, buf1)
        sems = (sem0, sem1)
        handles = [None] * F_

        def start(f):
            return pltpu.async_copy(
                table.at[idx_v.at[pl.ds(f * BW, BW)]], bufs[f % 2], sems[f % 2]
            )

        handles[0] = start(0)
        for f in range(F_):
            if f + 1 < F_:
                handles[f + 1] = start(f + 1)
            handles[f].wait()
            pltpu.sync_copy(bufs[f % 2], out.at[pl.ds(f * B_ + wid * BW, BW)])

    return sc_gather


BSB = 2048
NBB = B_ // BSB


def _tc_body(g_ref, w_ref, b_ref, gam_ref, bet_ref, o_ref):
    x = g_ref[...].astype(jnp.float32)
    h = jnp.dot(x, w_ref[...], preferred_element_type=jnp.float32) + b_ref[...]
    h = h / (1.0 + jnp.exp(-h))
    mu = jnp.mean(h, axis=1, keepdims=True)
    d = h - mu
    var = jnp.mean(d * d, axis=1, keepdims=True)
    y = d * lax.rsqrt(var + 1e-5)
    o_ref[...] = (y * gam_ref[...] + bet_ref[...]).reshape(1, BSB, DM_)


_tc_call = pl.pallas_call(
    _tc_body,
    grid=(F_, NBB),
    in_specs=[
        pl.BlockSpec((BSB, EMB_), lambda f, i: (f * NBB + i, 0)),
        pl.BlockSpec((EMB_, DM_), lambda f, i: (0, 0)),
        pl.BlockSpec((1, DM_), lambda f, i: (0, 0)),
        pl.BlockSpec((1, DM_), lambda f, i: (0, 0)),
        pl.BlockSpec((1, DM_), lambda f, i: (0, 0)),
    ],
    out_specs=pl.BlockSpec((1, BSB, DM_), lambda f, i: (f, i, 0)),
    out_shape=jax.ShapeDtypeStruct((F_, B_, DM_), jnp.float32),
)


def kernel(int_feats, missing_mask, emb_table, missing_embeddings, W, b, gamma, beta):
    featsT = int_feats.T
    table16 = emb_table.astype(jnp.bfloat16)
    g = _make_sc_gather()(table16, featsT)
    out3 = _tc_call(
        g, W, b.reshape(1, DM_), gamma.reshape(1, DM_), beta.reshape(1, DM_),
    )
    return jnp.transpose(out3, (1, 0, 2))


# speedup vs baseline: 1.6249x; 1.1828x over previous
import functools

import jax
import jax.numpy as jnp
from jax import lax
from jax.experimental import pallas as pl
from jax.experimental.pallas import tpu as pltpu
from jax.experimental.pallas import tpu_sc as plsc

B_ = 16384
F_ = 26
VOCAB_ = 100000
EMB_ = 32
DM_ = 128
ROWS = B_ * F_
NW = 32
BW = B_ // NW


@functools.cache
def _make_sc_gather():
    mesh = plsc.VectorSubcoreMesh(core_axis_name="c", subcore_axis_name="s")

    @functools.partial(
        pl.kernel,
        out_type=jax.ShapeDtypeStruct((ROWS, EMB_), jnp.float32),
        mesh=mesh,
        compiler_params=pltpu.CompilerParams(use_tc_tiling_on_sc=False),
        scratch_types=[
            pltpu.VMEM((F_, BW), jnp.int32),
            pltpu.VMEM((F_ * BW,), jnp.int32),
            pltpu.VMEM((BW, EMB_), jnp.float32),
            pltpu.VMEM((BW, EMB_), jnp.float32),
            pltpu.SemaphoreType.DMA,
            pltpu.SemaphoreType.DMA,
        ],
    )
    def sc_gather(table, featsT, out, feats_v, idx_v, buf0, buf1, sem0, sem1):
        wid = lax.axis_index("s") * 2 + lax.axis_index("c")
        pltpu.sync_copy(featsT.at[:, pl.ds(wid * BW, BW)], feats_v)

        def compute_idx(f, carry):
            off = f * VOCAB_
            for c in range(BW // 16):
                idx_v[pl.ds(f * BW + c * 16, 16)] = (
                    feats_v[f, pl.ds(c * 16, 16)] + off
                )
            return carry

        lax.fori_loop(0, F_, compute_idx, 0)

        bufs = (buf0, buf1)
        sems = (sem0, sem1)
        handles = [None] * F_

        def start(f):
            return pltpu.async_copy(
                table.at[idx_v.at[pl.ds(f * BW, BW)]], bufs[f % 2], sems[f % 2]
            )

        handles[0] = start(0)
        for f in range(F_):
            if f + 1 < F_:
                handles[f + 1] = start(f + 1)
            handles[f].wait()
            pltpu.sync_copy(bufs[f % 2], out.at[pl.ds(f * B_ + wid * BW, BW)])

    return sc_gather


BSB = 4096
NBB = B_ // BSB


def _tc_body(g_ref, w_ref, b_ref, gam_ref, bet_ref, o_ref):
    x = g_ref[...]
    h = jnp.dot(x, w_ref[...], preferred_element_type=jnp.float32) + b_ref[...]
    h = h / (1.0 + jnp.exp(-h))
    mu = jnp.mean(h, axis=1, keepdims=True)
    d = h - mu
    var = jnp.mean(d * d, axis=1, keepdims=True)
    y = d * lax.rsqrt(var + 1e-5)
    o_ref[...] = (y * gam_ref[...] + bet_ref[...]).reshape(1, BSB, DM_)


_tc_call = pl.pallas_call(
    _tc_body,
    grid=(F_, NBB),
    in_specs=[
        pl.BlockSpec((BSB, EMB_), lambda f, i: (f * NBB + i, 0)),
        pl.BlockSpec((EMB_, DM_), lambda f, i: (0, 0)),
        pl.BlockSpec((1, DM_), lambda f, i: (0, 0)),
        pl.BlockSpec((1, DM_), lambda f, i: (0, 0)),
        pl.BlockSpec((1, DM_), lambda f, i: (0, 0)),
    ],
    out_specs=pl.BlockSpec((1, BSB, DM_), lambda f, i: (f, i, 0)),
    out_shape=jax.ShapeDtypeStruct((F_, B_, DM_), jnp.float32),
)


def kernel(int_feats, missing_mask, emb_table, missing_embeddings, W, b, gamma, beta):
    featsT = int_feats.T
    g = _make_sc_gather()(emb_table, featsT)
    out3 = _tc_call(
        g, W, b.reshape(1, DM_), gamma.reshape(1, DM_), beta.reshape(1, DM_),
    )
    return jnp.transpose(out3, (1, 0, 2))


# speedup vs baseline: 1.6620x; 1.0228x over previous
import functools

import jax
import jax.numpy as jnp
from jax import lax
from jax.experimental import pallas as pl
from jax.experimental.pallas import tpu as pltpu
from jax.experimental.pallas import tpu_sc as plsc

B_ = 16384
F_ = 26
VOCAB_ = 100000
EMB_ = 32
DM_ = 128
ROWS = B_ * F_
NW = 32
BW = B_ // NW


@functools.cache
def _make_sc_gather():
    mesh = plsc.VectorSubcoreMesh(core_axis_name="c", subcore_axis_name="s")

    @functools.partial(
        pl.kernel,
        out_type=jax.ShapeDtypeStruct((ROWS, EMB_), jnp.float32),
        mesh=mesh,
        compiler_params=pltpu.CompilerParams(use_tc_tiling_on_sc=False),
        scratch_types=[
            pltpu.VMEM((F_, BW), jnp.int32),
            pltpu.VMEM((F_ * BW,), jnp.int32),
            pltpu.VMEM((BW, EMB_), jnp.float32),
            pltpu.VMEM((BW, EMB_), jnp.float32),
            pltpu.SemaphoreType.DMA,
            pltpu.SemaphoreType.DMA,
        ],
    )
    def sc_gather(table, featsT, out, feats_v, idx_v, buf0, buf1, sem0, sem1):
        wid = lax.axis_index("s") * 2 + lax.axis_index("c")
        pltpu.sync_copy(featsT.at[:, pl.ds(wid * BW, BW)], feats_v)

        def compute_idx(f, carry):
            off = f * VOCAB_
            for c in range(BW // 16):
                idx_v[pl.ds(f * BW + c * 16, 16)] = (
                    feats_v[f, pl.ds(c * 16, 16)] + off
                )
            return carry

        lax.fori_loop(0, F_, compute_idx, 0)

        bufs = (buf0, buf1)
        sems = (sem0, sem1)
        handles = [None] * F_

        def start(f):
            return pltpu.async_copy(
                table.at[idx_v.at[pl.ds(f * BW, BW)]], bufs[f % 2], sems[f % 2]
            )

        handles[0] = start(0)
        for f in range(F_):
            if f + 1 < F_:
                handles[f + 1] = start(f + 1)
            handles[f].wait()
            pltpu.sync_copy(bufs[f % 2], out.at[pl.ds(f * B_ + wid * BW, BW)])

    return sc_gather


BSB = 8192
NBB = B_ // BSB


def _tc_body(g_ref, w_ref, b_ref, gam_ref, bet_ref, o_ref):
    x = g_ref[...]
    h = jnp.dot(x, w_ref[...], preferred_element_type=jnp.float32) + b_ref[...]
    h = h / (1.0 + jnp.exp(-h))
    mu = jnp.mean(h, axis=1, keepdims=True)
    d = h - mu
    var = jnp.mean(d * d, axis=1, keepdims=True)
    y = d * lax.rsqrt(var + 1e-5)
    o_ref[...] = (y * gam_ref[...] + bet_ref[...]).reshape(1, BSB, DM_)


_tc_call = pl.pallas_call(
    _tc_body,
    grid=(F_, NBB),
    in_specs=[
        pl.BlockSpec((BSB, EMB_), lambda f, i: (f * NBB + i, 0)),
        pl.BlockSpec((EMB_, DM_), lambda f, i: (0, 0)),
        pl.BlockSpec((1, DM_), lambda f, i: (0, 0)),
        pl.BlockSpec((1, DM_), lambda f, i: (0, 0)),
        pl.BlockSpec((1, DM_), lambda f, i: (0, 0)),
    ],
    out_specs=pl.BlockSpec((1, BSB, DM_), lambda f, i: (f, i, 0)),
    out_shape=jax.ShapeDtypeStruct((F_, B_, DM_), jnp.float32),
)


def kernel(int_feats, missing_mask, emb_table, missing_embeddings, W, b, gamma, beta):
    featsT = int_feats.T
    g = _make_sc_gather()(emb_table, featsT)
    out3 = _tc_call(
        g, W, b.reshape(1, DM_), gamma.reshape(1, DM_), beta.reshape(1, DM_),
    )
    return jnp.transpose(out3, (1, 0, 2))


# speedup vs baseline: 1.6795x; 1.0105x over previous
import functools

import jax
import jax.numpy as jnp
from jax import lax
from jax.experimental import pallas as pl
from jax.experimental.pallas import tpu as pltpu
from jax.experimental.pallas import tpu_sc as plsc

B_ = 16384
F_ = 26
VOCAB_ = 100000
EMB_ = 32
DM_ = 128
ROWS = B_ * F_
NW = 32
BW = B_ // NW


@functools.cache
def _make_sc_gather():
    mesh = plsc.VectorSubcoreMesh(core_axis_name="c", subcore_axis_name="s")

    @functools.partial(
        pl.kernel,
        out_type=jax.ShapeDtypeStruct((ROWS, EMB_), jnp.float32),
        mesh=mesh,
        compiler_params=pltpu.CompilerParams(use_tc_tiling_on_sc=False),
        scratch_types=[
            pltpu.VMEM((F_, BW), jnp.int32),
            pltpu.VMEM((F_ * BW,), jnp.int32),
            pltpu.VMEM((BW, EMB_), jnp.float32),
            pltpu.VMEM((BW, EMB_), jnp.float32),
            pltpu.SemaphoreType.DMA,
            pltpu.SemaphoreType.DMA,
        ],
    )
    def sc_gather(table, featsT, out, feats_v, idx_v, buf0, buf1, sem0, sem1):
        wid = lax.axis_index("s") * 2 + lax.axis_index("c")
        pltpu.sync_copy(featsT.at[:, pl.ds(wid * BW, BW)], feats_v)

        def compute_idx(f, carry):
            off = f * VOCAB_
            for c in range(BW // 16):
                idx_v[pl.ds(f * BW + c * 16, 16)] = (
                    feats_v[f, pl.ds(c * 16, 16)] + off
                )
            return carry

        lax.fori_loop(0, F_, compute_idx, 0)

        bufs = (buf0, buf1)
        sems = (sem0, sem1)
        handles = [None] * F_

        def start(f):
            return pltpu.async_copy(
                table.at[idx_v.at[pl.ds(f * BW, BW)]], bufs[f % 2], sems[f % 2]
            )

        handles[0] = start(0)
        for f in range(F_):
            if f + 1 < F_:
                handles[f + 1] = start(f + 1)
            handles[f].wait()
            pltpu.sync_copy(bufs[f % 2], out.at[pl.ds(f * B_ + wid * BW, BW)])

    return sc_gather


BSB = 16384
NBB = B_ // BSB


def _tc_body(g_ref, w_ref, b_ref, gam_ref, bet_ref, o_ref):
    x = g_ref[...]
    h = jnp.dot(x, w_ref[...], preferred_element_type=jnp.float32) + b_ref[...]
    h = h / (1.0 + jnp.exp(-h))
    mu = jnp.mean(h, axis=1, keepdims=True)
    d = h - mu
    var = jnp.mean(d * d, axis=1, keepdims=True)
    y = d * lax.rsqrt(var + 1e-5)
    o_ref[...] = (y * gam_ref[...] + bet_ref[...]).reshape(1, BSB, DM_)


_tc_call = pl.pallas_call(
    _tc_body,
    grid=(F_, NBB),
    in_specs=[
        pl.BlockSpec((BSB, EMB_), lambda f, i: (f * NBB + i, 0)),
        pl.BlockSpec((EMB_, DM_), lambda f, i: (0, 0)),
        pl.BlockSpec((1, DM_), lambda f, i: (0, 0)),
        pl.BlockSpec((1, DM_), lambda f, i: (0, 0)),
        pl.BlockSpec((1, DM_), lambda f, i: (0, 0)),
    ],
    out_specs=pl.BlockSpec((1, BSB, DM_), lambda f, i: (f, i, 0)),
    out_shape=jax.ShapeDtypeStruct((F_, B_, DM_), jnp.float32),
)


def kernel(int_feats, missing_mask, emb_table, missing_embeddings, W, b, gamma, beta):
    featsT = int_feats.T
    g = _make_sc_gather()(emb_table, featsT)
    out3 = _tc_call(
        g, W, b.reshape(1, DM_), gamma.reshape(1, DM_), beta.reshape(1, DM_),
    )
    return jnp.transpose(out3, (1, 0, 2))
